# Initial kernel scaffold; baseline (speedup 1.0000x reference)
import jax
import jax.numpy as jnp
from jax.experimental import pallas as pl


def kernel(indices, table):
    raise NotImplementedError("write your pallas kernel here")


# speedup vs baseline: 3.1088x; 3.1088x over previous
import functools

import jax
import jax.numpy as jnp
from jax import lax
from jax.experimental import pallas as pl
from jax.experimental.pallas import tpu as pltpu
from jax.experimental.pallas import tpu_sc as plsc

_NUM_CORES = 2
_NUM_SUBCORES = 16
_NW = _NUM_CORES * _NUM_SUBCORES
_CHUNK = 128
_B, _V = 4096, 102
_D = 64
_TOTAL = _B * _V
_PER_W = _TOTAL // _NW
_NCHUNK = _PER_W // _CHUNK


def _sc_body(idx_hbm, table_hbm, out_hbm, idx_v, rows_v, gsem):
    wid = lax.axis_index("s") * _NUM_CORES + lax.axis_index("c")
    base = wid * _PER_W
    pltpu.sync_copy(idx_hbm.at[wid], idx_v)

    def chunk(c, carry):
        pltpu.async_copy(table_hbm.at[idx_v.at[c]], rows_v, gsem).wait()
        pltpu.sync_copy(rows_v, out_hbm.at[pl.ds(base + c * _CHUNK, _CHUNK)])
        return carry

    lax.fori_loop(0, _NCHUNK, chunk, 0)


@jax.jit
def _lookup(idx_resh, table):
    mesh = plsc.VectorSubcoreMesh(core_axis_name="c", subcore_axis_name="s")
    f = pl.kernel(
        _sc_body,
        out_type=jax.ShapeDtypeStruct((_TOTAL, _D), jnp.float32),
        mesh=mesh,
        scratch_types=[
            pltpu.VMEM((_NCHUNK, _CHUNK), jnp.int32),
            pltpu.VMEM((_CHUNK, _D), jnp.float32),
            pltpu.SemaphoreType.DMA,
        ],
        compiler_params=pltpu.CompilerParams(use_tc_tiling_on_sc=False),
    )
    return f(idx_resh, table)


def kernel(indices, table):
    idx_resh = indices.reshape(_NW, _NCHUNK, _CHUNK)
    out = _lookup(idx_resh, table)
    return out.reshape(_B, _V, _D)


# speedup vs baseline: 3.1341x; 1.0081x over previous
import jax
import jax.numpy as jnp
from jax import lax
from jax.experimental import pallas as pl
from jax.experimental.pallas import tpu as pltpu
from jax.experimental.pallas import tpu_sc as plsc

_NUM_CORES = 2
_NUM_SUBCORES = 16
_NW = _NUM_CORES * _NUM_SUBCORES
_IDXW = 256
_B, _V = 4096, 102
_D = 64
_TOTAL = _B * _V
_PER_W = _TOTAL // _NW
_NG = _PER_W // _IDXW
_NBUF = 3


def _sc_body(idx_hbm, table_hbm, out_hbm, idx_v, bufs, gsems, ssems):
    wid = lax.axis_index("s") * _NUM_CORES + lax.axis_index("c")
    base = wid * _PER_W
    pltpu.sync_copy(idx_hbm.at[wid], idx_v)

    def out_slice(g):
        return out_hbm.at[pl.ds(base + g * _IDXW, _IDXW)]

    def fire(g, b):
        pltpu.async_copy(table_hbm.at[idx_v.at[g]], bufs[b], gsems[b])

    def wait_gather(g, b):
        pltpu.make_async_copy(table_hbm.at[idx_v.at[g]], bufs[b], gsems[b]).wait()

    def store(g, b):
        pltpu.async_copy(bufs[b], out_slice(g), ssems[b])

    def wait_store(g, b):
        pltpu.make_async_copy(bufs[b], out_slice(g), ssems[b]).wait()

    fire(0, 0)
    fire(1, 1)

    fire(2, 2)
    wait_gather(0, 0)
    store(0, 0)

    for g in (1, 2):
        wait_store(g - 1, (g - 1) % _NBUF)
        fire(g + 2, (g + 2) % _NBUF)
        wait_gather(g, g % _NBUF)
        store(g, g % _NBUF)

    def body(t, carry):
        for b in range(_NBUF):
            g = t * _NBUF + b
            wait_store(g - 1, (b - 1) % _NBUF)
            fire(g + 2, (b + 2) % _NBUF)
            wait_gather(g, b)
            store(g, b)
        return carry

    lax.fori_loop(1, (_NG - 6) // _NBUF + 1, body, 0)

    g = _NG - 3
    wait_store(g - 1, (g - 1) % _NBUF)
    fire(g + 2, (g + 2) % _NBUF)
    wait_gather(g, g % _NBUF)
    store(g, g % _NBUF)
    for g in (_NG - 2, _NG - 1):
        wait_store(g - 1, (g - 1) % _NBUF)
        wait_gather(g, g % _NBUF)
        store(g, g % _NBUF)
    wait_store(_NG - 1, (_NG - 1) % _NBUF)


@jax.jit
def _lookup(idx_resh, table):
    mesh = plsc.VectorSubcoreMesh(core_axis_name="c", subcore_axis_name="s")
    f = pl.kernel(
        _sc_body,
        out_type=jax.ShapeDtypeStruct((_TOTAL, _D), jnp.float32),
        mesh=mesh,
        scratch_types=[
            pltpu.VMEM((_NG, _IDXW), jnp.int32),
            [pltpu.VMEM((_IDXW, _D), jnp.float32) for _ in range(_NBUF)],
            [pltpu.SemaphoreType.DMA for _ in range(_NBUF)],
            [pltpu.SemaphoreType.DMA for _ in range(_NBUF)],
        ],
        compiler_params=pltpu.CompilerParams(use_tc_tiling_on_sc=False),
    )
    return f(idx_resh, table)


def kernel(indices, table):
    idx_resh = indices.reshape(_NW, _NG, _IDXW)
    out = _lookup(idx_resh, table)
    return out.reshape(_B, _V, _D)
